# baseline (device time: 22810 ns/iter reference)
import jax
import jax.numpy as jnp
from jax import lax
from jax.experimental import pallas as pl
from jax.experimental.pallas import tpu as pltpu

N_DEV = 32


def kernel(t):
    m, n = t.shape
    rows = m // N_DEV

    def body(
        t_ref,
        out_ref,
        stage1,
        comm1,
        stage2,
        comm2,
        send1,
        recv1,
        send2,
        recv2,
    ):
        my = lax.axis_index("i")

        barrier_sem = pltpu.get_barrier_semaphore()
        for o in range(1, N_DEV):
            pl.semaphore_signal(
                barrier_sem,
                inc=1,
                device_id=((my + o) % N_DEV,),
                device_id_type=pl.DeviceIdType.MESH,
            )

        stage1[...] = t_ref[...].astype(jnp.bfloat16)
        pl.semaphore_wait(barrier_sem, N_DEV - 1)
        rdma1 = []
        for o in range(1, N_DEV):
            d = (my + o) % N_DEV
            r = pltpu.make_async_remote_copy(
                src_ref=stage1.at[pl.ds(d * rows, rows), :],
                dst_ref=comm1.at[o],
                send_sem=send1.at[o],
                recv_sem=recv1.at[o],
                device_id=(d,),
                device_id_type=pl.DeviceIdType.MESH,
            )
            r.start()
            rdma1.append(r)

        acc = t_ref[pl.ds(my * rows, rows), :]
        for o in range(1, N_DEV):
            rdma1[o - 1].wait_recv()
            acc = acc + comm1[o].astype(jnp.float32)

        rpos = jnp.maximum(acc, 0.0)
        y = jnp.tanh(acc) * acc * acc + rpos * rpos * rpos
        out_ref[pl.ds(my * rows, rows), :] = y
        stage2[...] = y.astype(jnp.bfloat16)

        rdma2 = []
        for o in range(1, N_DEV):
            d = (my + o) % N_DEV
            r = pltpu.make_async_remote_copy(
                src_ref=stage2,
                dst_ref=comm2.at[o],
                send_sem=send2.at[o],
                recv_sem=recv2.at[o],
                device_id=(d,),
                device_id_type=pl.DeviceIdType.MESH,
            )
            r.start()
            rdma2.append(r)

        for o in range(1, N_DEV):
            rdma2[o - 1].wait_recv()
            s = (my - o) % N_DEV
            out_ref[pl.ds(s * rows, rows), :] = comm2[o].astype(jnp.float32)

        for r in rdma1:
            r.wait_send()
        for r in rdma2:
            r.wait_send()

    return pl.pallas_call(
        body,
        out_shape=jax.ShapeDtypeStruct((m, n), jnp.float32),
        in_specs=[pl.BlockSpec(memory_space=pltpu.VMEM)],
        out_specs=pl.BlockSpec(memory_space=pltpu.VMEM),
        scratch_shapes=[
            pltpu.VMEM((m, n), jnp.bfloat16),
            pltpu.VMEM((N_DEV, rows, n), jnp.bfloat16),
            pltpu.VMEM((rows, n), jnp.bfloat16),
            pltpu.VMEM((N_DEV, rows, n), jnp.bfloat16),
            pltpu.SemaphoreType.DMA((N_DEV,)),
            pltpu.SemaphoreType.DMA((N_DEV,)),
            pltpu.SemaphoreType.DMA((N_DEV,)),
            pltpu.SemaphoreType.DMA((N_DEV,)),
        ],
        compiler_params=pltpu.CompilerParams(collective_id=0),
    )(t)


# device time: 18996 ns/iter; 1.2008x vs baseline; 1.2008x over previous
import jax
import jax.numpy as jnp
from jax import lax
from jax.experimental import pallas as pl
from jax.experimental.pallas import tpu as pltpu

N_DEV = 32


def kernel(t):
    m, n = t.shape
    rows = m // N_DEV

    def body(
        t_ref,
        out_ref,
        stage1,
        comm1,
        stage2,
        comm2,
        stage_out,
        send1,
        recv1,
        send2,
        recv2,
        out_sems,
    ):
        my = lax.axis_index("i")

        barrier_sem = pltpu.get_barrier_semaphore()
        for o in range(1, N_DEV):
            pl.semaphore_signal(
                barrier_sem,
                inc=1,
                device_id=((my + o) % N_DEV,),
                device_id_type=pl.DeviceIdType.MESH,
            )

        stage1[...] = t_ref[...].astype(jnp.bfloat16)
        pl.semaphore_wait(barrier_sem, N_DEV - 1)
        rdma1 = []
        for o in range(1, N_DEV):
            d = (my + o) % N_DEV
            r = pltpu.make_async_remote_copy(
                src_ref=stage1.at[pl.ds(d * rows, rows), :],
                dst_ref=comm1.at[o],
                send_sem=send1.at[o],
                recv_sem=recv1.at[o],
                device_id=(d,),
                device_id_type=pl.DeviceIdType.MESH,
            )
            r.start()
            rdma1.append(r)

        acc = t_ref[pl.ds(my * rows, rows), :]
        for o in range(1, N_DEV):
            rdma1[o - 1].wait_recv()
            acc = acc + comm1[o].astype(jnp.float32)

        rpos = jnp.maximum(acc, 0.0)
        y = jnp.tanh(acc) * acc * acc + rpos * rpos * rpos
        stage_out[0] = y
        stage2[...] = y.astype(jnp.bfloat16)
        copies = []
        c = pltpu.make_async_copy(
            stage_out.at[0], out_ref.at[pl.ds(my * rows, rows), :], out_sems.at[0]
        )
        c.start()
        copies.append(c)

        rdma2 = []
        for o in range(1, N_DEV):
            d = (my + o) % N_DEV
            r = pltpu.make_async_remote_copy(
                src_ref=stage2,
                dst_ref=comm2.at[o],
                send_sem=send2.at[o],
                recv_sem=recv2.at[o],
                device_id=(d,),
                device_id_type=pl.DeviceIdType.MESH,
            )
            r.start()
            rdma2.append(r)

        for o in range(1, N_DEV):
            rdma2[o - 1].wait_recv()
            s = (my - o) % N_DEV
            stage_out[o] = comm2[o].astype(jnp.float32)
            c = pltpu.make_async_copy(
                stage_out.at[o], out_ref.at[pl.ds(s * rows, rows), :], out_sems.at[o]
            )
            c.start()
            copies.append(c)

        for c in copies:
            c.wait()
        for r in rdma1:
            r.wait_send()
        for r in rdma2:
            r.wait_send()

    return pl.pallas_call(
        body,
        out_shape=jax.ShapeDtypeStruct((m, n), jnp.float32),
        in_specs=[pl.BlockSpec(memory_space=pltpu.VMEM)],
        out_specs=pl.BlockSpec(memory_space=pl.ANY),
        scratch_shapes=[
            pltpu.VMEM((m, n), jnp.bfloat16),
            pltpu.VMEM((N_DEV, rows, n), jnp.bfloat16),
            pltpu.VMEM((rows, n), jnp.bfloat16),
            pltpu.VMEM((N_DEV, rows, n), jnp.bfloat16),
            pltpu.VMEM((N_DEV, rows, n), jnp.float32),
            pltpu.SemaphoreType.DMA((N_DEV,)),
            pltpu.SemaphoreType.DMA((N_DEV,)),
            pltpu.SemaphoreType.DMA((N_DEV,)),
            pltpu.SemaphoreType.DMA((N_DEV,)),
            pltpu.SemaphoreType.DMA((N_DEV,)),
        ],
        compiler_params=pltpu.CompilerParams(collective_id=0),
    )(t)
